# Initial kernel scaffold; baseline (speedup 1.0000x reference)
import jax
import jax.numpy as jnp
from jax.experimental import pallas as pl


def kernel(x):
    raise NotImplementedError("write your pallas kernel here")


# speedup vs baseline: 1.0963x; 1.0963x over previous
import functools

import jax
import jax.numpy as jnp
from jax.experimental import pallas as pl

_MASK_RATIO = 0.5


def _mask_kernel(noise_ref, x_ref, out_ref, mask_ref, *, n, num_masked):
    a = noise_ref[0]
    ai = a[:, :, None]
    aj = a[:, None, :]
    ii = jax.lax.broadcasted_iota(jnp.int32, (1, n, n), 1)
    jj = jax.lax.broadcasted_iota(jnp.int32, (1, n, n), 2)
    before = (aj < ai) | ((aj == ai) & (jj < ii))
    rank = jnp.sum(before.astype(jnp.int32), axis=2)
    masked = rank < num_masked
    mask_ref[...] = masked.astype(jnp.int32)[None]
    out_ref[...] = jnp.where(masked[:, :, None, None], 0.0, x_ref[...])


def kernel(x):
    b, n, p, d = x.shape
    num_masked = int(_MASK_RATIO * n)
    noise = jax.random.uniform(jax.random.key(1), (b, n), dtype=jnp.float32)
    noise3 = noise.reshape(b, 1, n)
    out, mask3 = pl.pallas_call(
        functools.partial(_mask_kernel, n=n, num_masked=num_masked),
        grid=(b,),
        in_specs=[
            pl.BlockSpec((1, 1, n), lambda i: (i, 0, 0)),
            pl.BlockSpec((1, n, p, d), lambda i: (i, 0, 0, 0)),
        ],
        out_specs=[
            pl.BlockSpec((1, n, p, d), lambda i: (i, 0, 0, 0)),
            pl.BlockSpec((1, 1, n), lambda i: (i, 0, 0)),
        ],
        out_shape=[
            jax.ShapeDtypeStruct((b, n, p, d), x.dtype),
            jax.ShapeDtypeStruct((b, 1, n), jnp.int32),
        ],
    )(noise3, x)
    return out, mask3.reshape(b, n).astype(bool)
